# Initial kernel scaffold; baseline (speedup 1.0000x reference)
import jax
import jax.numpy as jnp
from jax.experimental import pallas as pl


def kernel(aa_indices, biochem_features, table, Wp, bp, Wf, bf, gamma, beta):
    raise NotImplementedError("write your pallas kernel here")


# speedup vs baseline: 3.9568x; 3.9568x over previous
import jax
import jax.numpy as jnp
from jax import lax
from jax.experimental import pallas as pl
from jax.experimental.pallas import tpu as pltpu

_B, _S = 128, 1024
_VOCAB, _DIM, _PAD, _BIO = 25, 256, 20, 5
_N = _B * _S
_T = 2048


def _fold_body(table_ref, wf_ref, wp_ref, bp_ref, bf_ref, w1_ref, w2_ref, b2_ref):
    tbl = table_ref[...]
    row = lax.broadcasted_iota(jnp.int32, tbl.shape, 0)
    tbl = jnp.where(row == _PAD, 0.0, tbl)
    wf_top = wf_ref[0:_DIM, :]
    wf_bot = wf_ref[_DIM:2 * _DIM, :]
    w1_ref[...] = jnp.dot(tbl, wf_top, preferred_element_type=jnp.float32)
    w2_ref[...] = jnp.dot(wp_ref[...], wf_bot, preferred_element_type=jnp.float32)
    b2_ref[...] = jnp.dot(bp_ref[...], wf_bot, preferred_element_type=jnp.float32) + bf_ref[...]


def _main_body(idx_ref, bio_ref, w1_ref, w2_ref, b2_ref, g_ref, bt_ref, out_ref):
    idx = idx_ref[...]
    oh = (idx == lax.broadcasted_iota(jnp.int32, (_T, 32), 1)).astype(jnp.float32)
    e = jnp.dot(oh, w1_ref[...], preferred_element_type=jnp.float32)
    e = e + jnp.dot(bio_ref[...], w2_ref[...], preferred_element_type=jnp.float32)
    e = e + b2_ref[...]
    m = jnp.mean(e, axis=1, keepdims=True)
    c = e - m
    v = jnp.mean(c * c, axis=1, keepdims=True)
    out_ref[...] = c * lax.rsqrt(v + 1e-5) * g_ref[...] + bt_ref[...]


def kernel(aa_indices, biochem_features, table, Wp, bp, Wf, bf, gamma, beta):
    idx = aa_indices.astype(jnp.int32).reshape(_N, 1)
    bio = jnp.pad(biochem_features.reshape(_N, _BIO), ((0, 0), (0, 8 - _BIO)))
    table_pad = jnp.pad(table, ((0, 32 - _VOCAB), (0, 0)))
    wp_pad = jnp.pad(Wp, ((0, 8 - _BIO), (0, 0)))
    bp2 = bp.reshape(1, _DIM)
    bf2 = bf.reshape(1, _DIM)
    g2 = gamma.reshape(1, _DIM)
    bt2 = beta.reshape(1, _DIM)

    w1, w2, b2 = pl.pallas_call(
        _fold_body,
        out_shape=(
            jax.ShapeDtypeStruct((32, _DIM), jnp.float32),
            jax.ShapeDtypeStruct((8, _DIM), jnp.float32),
            jax.ShapeDtypeStruct((1, _DIM), jnp.float32),
        ),
    )(table_pad, Wf, wp_pad, bp2, bf2)

    nb = _N // _T
    out = pl.pallas_call(
        _main_body,
        grid=(nb,),
        in_specs=[
            pl.BlockSpec((_T, 1), lambda i: (i, 0)),
            pl.BlockSpec((_T, 8), lambda i: (i, 0)),
            pl.BlockSpec((32, _DIM), lambda i: (0, 0)),
            pl.BlockSpec((8, _DIM), lambda i: (0, 0)),
            pl.BlockSpec((1, _DIM), lambda i: (0, 0)),
            pl.BlockSpec((1, _DIM), lambda i: (0, 0)),
            pl.BlockSpec((1, _DIM), lambda i: (0, 0)),
        ],
        out_specs=pl.BlockSpec((_T, _DIM), lambda i: (i, 0)),
        out_shape=jax.ShapeDtypeStruct((_N, _DIM), jnp.float32),
        compiler_params=pltpu.CompilerParams(
            dimension_semantics=("arbitrary",),
        ),
    )(idx, bio, w1, w2, b2, g2, bt2)
    return out.reshape(_B, _S, _DIM)


# speedup vs baseline: 4.4361x; 1.1211x over previous
import jax
import jax.numpy as jnp
from jax import lax
from jax.experimental import pallas as pl
from jax.experimental.pallas import tpu as pltpu

_B, _S = 128, 1024
_VOCAB, _DIM, _PAD, _BIO = 25, 256, 20, 5
_N = _B * _S
_T = 8192


def _fold_body(table_ref, wf_ref, wp_ref, bp_ref, bf_ref, w1_ref, w2_ref, b2_ref):
    tbl = table_ref[...]
    row = lax.broadcasted_iota(jnp.int32, tbl.shape, 0)
    tbl = jnp.where(row == _PAD, 0.0, tbl)
    wf_top = wf_ref[0:_DIM, :]
    wf_bot = wf_ref[_DIM:2 * _DIM, :]
    w1_ref[...] = jnp.dot(tbl, wf_top, preferred_element_type=jnp.float32)
    w2_ref[...] = jnp.dot(wp_ref[...], wf_bot, preferred_element_type=jnp.float32)
    b2_ref[...] = jnp.dot(bp_ref[...], wf_bot, preferred_element_type=jnp.float32) + bf_ref[...]


def _main_body(idx_ref, bio_ref, w1_ref, w2_ref, b2_ref, g_ref, bt_ref, out_ref):
    idx = idx_ref[...]
    oh = (idx == lax.broadcasted_iota(jnp.int32, (_T, 32), 1)).astype(jnp.float32)
    e = jnp.dot(oh, w1_ref[...], preferred_element_type=jnp.float32)
    e = e + jnp.dot(bio_ref[...], w2_ref[...], preferred_element_type=jnp.float32)
    e = e + b2_ref[...]
    m = jnp.mean(e, axis=1, keepdims=True)
    c = e - m
    v = jnp.mean(c * c, axis=1, keepdims=True)
    out_ref[...] = c * lax.rsqrt(v + 1e-5) * g_ref[...] + bt_ref[...]


def kernel(aa_indices, biochem_features, table, Wp, bp, Wf, bf, gamma, beta):
    idx = aa_indices.astype(jnp.int32).reshape(_N, 1)
    bio = jnp.pad(biochem_features.reshape(_N, _BIO), ((0, 0), (0, 8 - _BIO)))
    table_pad = jnp.pad(table, ((0, 32 - _VOCAB), (0, 0)))
    wp_pad = jnp.pad(Wp, ((0, 8 - _BIO), (0, 0)))
    bp2 = bp.reshape(1, _DIM)
    bf2 = bf.reshape(1, _DIM)
    g2 = gamma.reshape(1, _DIM)
    bt2 = beta.reshape(1, _DIM)

    w1, w2, b2 = pl.pallas_call(
        _fold_body,
        out_shape=(
            jax.ShapeDtypeStruct((32, _DIM), jnp.float32),
            jax.ShapeDtypeStruct((8, _DIM), jnp.float32),
            jax.ShapeDtypeStruct((1, _DIM), jnp.float32),
        ),
    )(table_pad, Wf, wp_pad, bp2, bf2)

    nb = _N // _T
    out = pl.pallas_call(
        _main_body,
        grid=(nb,),
        in_specs=[
            pl.BlockSpec((_T, 1), lambda i: (i, 0)),
            pl.BlockSpec((_T, 8), lambda i: (i, 0)),
            pl.BlockSpec((32, _DIM), lambda i: (0, 0)),
            pl.BlockSpec((8, _DIM), lambda i: (0, 0)),
            pl.BlockSpec((1, _DIM), lambda i: (0, 0)),
            pl.BlockSpec((1, _DIM), lambda i: (0, 0)),
            pl.BlockSpec((1, _DIM), lambda i: (0, 0)),
        ],
        out_specs=pl.BlockSpec((_T, _DIM), lambda i: (i, 0)),
        out_shape=jax.ShapeDtypeStruct((_N, _DIM), jnp.float32),
        compiler_params=pltpu.CompilerParams(
            dimension_semantics=("arbitrary",),
        ),
    )(idx, bio, w1, w2, b2, g2, bt2)
    return out.reshape(_B, _S, _DIM)


# speedup vs baseline: 4.5465x; 1.0249x over previous
import jax
import jax.numpy as jnp
from jax import lax
from jax.experimental import pallas as pl
from jax.experimental.pallas import tpu as pltpu

_B, _S = 128, 1024
_VOCAB, _DIM, _PAD, _BIO = 25, 256, 20, 5
_N = _B * _S
_T = 8192


def _fold_body(table_ref, wf_ref, wp_ref, bp_ref, bf_ref, w1_ref, w2_ref, b2_ref):
    tbl = table_ref[...]
    row = lax.broadcasted_iota(jnp.int32, tbl.shape, 0)
    tbl = jnp.where(row == _PAD, 0.0, tbl)
    wf_top = wf_ref[0:_DIM, :]
    wf_bot = wf_ref[_DIM:2 * _DIM, :]
    w1 = jnp.dot(tbl, wf_top, preferred_element_type=jnp.float32)
    w2 = jnp.dot(wp_ref[...], wf_bot, preferred_element_type=jnp.float32)
    b2 = jnp.dot(bp_ref[...], wf_bot, preferred_element_type=jnp.float32) + bf_ref[...]
    w1_ref[...] = w1 - jnp.mean(w1, axis=1, keepdims=True)
    w2_ref[...] = w2 - jnp.mean(w2, axis=1, keepdims=True)
    b2_ref[...] = b2 - jnp.mean(b2, axis=1, keepdims=True)


def _main_body(idx_ref, bio_ref, w1_ref, w2_ref, b2_ref, g_ref, bt_ref, out_ref):
    idx = idx_ref[...]
    oh = (idx == lax.broadcasted_iota(jnp.int32, (_T, 32), 1)).astype(jnp.float32)
    c = jnp.dot(oh, w1_ref[...], preferred_element_type=jnp.float32)
    c = c + jnp.dot(bio_ref[...], w2_ref[...], preferred_element_type=jnp.float32)
    c = c + b2_ref[...]
    v = jnp.mean(c * c, axis=1, keepdims=True)
    out_ref[...] = c * lax.rsqrt(v + 1e-5) * g_ref[...] + bt_ref[...]


def kernel(aa_indices, biochem_features, table, Wp, bp, Wf, bf, gamma, beta):
    idx = aa_indices.astype(jnp.int32).reshape(_N, 1)
    bio = jnp.pad(biochem_features.reshape(_N, _BIO), ((0, 0), (0, 8 - _BIO)))
    table_pad = jnp.pad(table, ((0, 32 - _VOCAB), (0, 0)))
    wp_pad = jnp.pad(Wp, ((0, 8 - _BIO), (0, 0)))
    bp2 = bp.reshape(1, _DIM)
    bf2 = bf.reshape(1, _DIM)
    g2 = gamma.reshape(1, _DIM)
    bt2 = beta.reshape(1, _DIM)

    w1, w2, b2 = pl.pallas_call(
        _fold_body,
        out_shape=(
            jax.ShapeDtypeStruct((32, _DIM), jnp.float32),
            jax.ShapeDtypeStruct((8, _DIM), jnp.float32),
            jax.ShapeDtypeStruct((1, _DIM), jnp.float32),
        ),
    )(table_pad, Wf, wp_pad, bp2, bf2)

    nb = _N // _T
    out = pl.pallas_call(
        _main_body,
        grid=(nb,),
        in_specs=[
            pl.BlockSpec((_T, 1), lambda i: (i, 0)),
            pl.BlockSpec((_T, 8), lambda i: (i, 0)),
            pl.BlockSpec((32, _DIM), lambda i: (0, 0)),
            pl.BlockSpec((8, _DIM), lambda i: (0, 0)),
            pl.BlockSpec((1, _DIM), lambda i: (0, 0)),
            pl.BlockSpec((1, _DIM), lambda i: (0, 0)),
            pl.BlockSpec((1, _DIM), lambda i: (0, 0)),
        ],
        out_specs=pl.BlockSpec((_T, _DIM), lambda i: (i, 0)),
        out_shape=jax.ShapeDtypeStruct((_N, _DIM), jnp.float32),
        compiler_params=pltpu.CompilerParams(
            dimension_semantics=("arbitrary",),
        ),
    )(idx, bio, w1, w2, b2, g2, bt2)
    return out.reshape(_B, _S, _DIM)
